# Initial kernel scaffold; baseline (speedup 1.0000x reference)
import jax
import jax.numpy as jnp
from jax.experimental import pallas as pl


def kernel(symbols, relations, params):
    raise NotImplementedError("write your pallas kernel here")


# speedup vs baseline: 1.3947x; 1.3947x over previous
import functools

import jax
import jax.numpy as jnp
from jax.experimental import pallas as pl
from jax.experimental.pallas import tpu as pltpu

N_NODES = 10000
N_EDGES = 2048
DIM = 128


def _edge_mlp_body(rows_ref, rel_ref, w1a, w1b, b1, w2, b2, out_ref):
    xs = rows_ref[:N_EDGES, :]
    xt = rows_ref[N_EDGES:, :]
    h = jnp.dot(xs, w1a[...], preferred_element_type=jnp.float32)
    h = h + jnp.dot(xt, w1b[...], preferred_element_type=jnp.float32)
    h = jnp.maximum(h + b1[...], 0.0)
    msg = jnp.dot(h, w2[...], preferred_element_type=jnp.float32) + b2[...]
    src = rel_ref[:, 0:1]
    tgt = rel_ref[:, 2:3]
    valid = ((src < N_NODES) & (tgt < N_NODES)).astype(jnp.float32)
    out_ref[...] = msg * valid


def _edge_mlp(rows, relations, W1, b1, W2, b2):
    return pl.pallas_call(
        _edge_mlp_body,
        out_shape=jax.ShapeDtypeStruct((N_EDGES, DIM), jnp.float32),
    )(rows, relations, W1[:DIM], W1[DIM:], b1.reshape(1, DIM), W2,
      b2.reshape(1, DIM))


def _gru_body(agg_ref, x_ref, wih, bih, whh, bhh, out_ref):
    gi = jnp.dot(agg_ref[...], wih[...],
                 preferred_element_type=jnp.float32) + bih[...]
    gh = jnp.dot(x_ref[...], whh[...],
                 preferred_element_type=jnp.float32) + bhh[...]
    r = jax.nn.sigmoid(gi[:, :DIM] + gh[:, :DIM])
    z = jax.nn.sigmoid(gi[:, DIM:2 * DIM] + gh[:, DIM:2 * DIM])
    n = jnp.tanh(gi[:, 2 * DIM:] + r * gh[:, 2 * DIM:])
    out_ref[...] = (1.0 - z) * n + z * x_ref[...]


def _gru(agg, x, Wih, bih, Whh, bhh):
    R = 1000
    full = lambda i: (0, 0)
    return pl.pallas_call(
        _gru_body,
        grid=(N_NODES // R,),
        in_specs=[
            pl.BlockSpec((R, DIM), lambda i: (i, 0)),
            pl.BlockSpec((R, DIM), lambda i: (i, 0)),
            pl.BlockSpec((DIM, 3 * DIM), full),
            pl.BlockSpec((1, 3 * DIM), full),
            pl.BlockSpec((DIM, 3 * DIM), full),
            pl.BlockSpec((1, 3 * DIM), full),
        ],
        out_specs=pl.BlockSpec((R, DIM), lambda i: (i, 0)),
        out_shape=jax.ShapeDtypeStruct((N_NODES, DIM), jnp.float32),
    )(agg, x, Wih, bih.reshape(1, -1), Whh, bhh.reshape(1, -1))


def _layer_norm(h, g, b):
    mu = jnp.mean(h, axis=-1, keepdims=True)
    var = jnp.mean((h - mu) ** 2, axis=-1, keepdims=True)
    return (h - mu) * jax.lax.rsqrt(var + 1e-5) * g + b


def _readout_body(sym_ref, x_ref, d1, db1, g1, c1, d2, db2, g2, c2, d3, db3,
                  out_ref):
    agg = jnp.dot(sym_ref[...], x_ref[...], preferred_element_type=jnp.float32)
    h = jnp.dot(agg, d1[...], preferred_element_type=jnp.float32) + db1[...]
    h = jnp.maximum(_layer_norm(h, g1[...], c1[...]), 0.0)
    h = jnp.dot(h, d2[...], preferred_element_type=jnp.float32) + db2[...]
    h = jnp.maximum(_layer_norm(h, g2[...], c2[...]), 0.0)
    out_ref[...] = jnp.dot(h, d3[...],
                           preferred_element_type=jnp.float32) + db3[...]


def _readout(symbols, x, p):
    B = symbols.shape[0]
    return pl.pallas_call(
        _readout_body,
        out_shape=jax.ShapeDtypeStruct((B, DIM), jnp.float32),
    )(symbols, x,
      p["D1"], p["db1"].reshape(1, -1), p["ln1_g"].reshape(1, -1),
      p["ln1_b"].reshape(1, -1),
      p["D2"], p["db2"].reshape(1, -1), p["ln2_g"].reshape(1, -1),
      p["ln2_b"].reshape(1, -1),
      p["D3"], p["db3"].reshape(1, -1))


def kernel(symbols, relations, params):
    p = params
    x = p["emb"]
    src = relations[:, 0]
    tgt = relations[:, 2]
    idx = jnp.concatenate([src, tgt], axis=0)
    for i in range(3):
        rows = x[idx]
        msg = _edge_mlp(rows, relations, p[f"g{i}_W1"], p[f"g{i}_b1"],
                        p[f"g{i}_W2"], p[f"g{i}_b2"])
        agg = jnp.zeros((N_NODES, DIM), jnp.float32).at[tgt].add(msg)
        x = _gru(agg, x, p[f"g{i}_Wih"], p[f"g{i}_bih"], p[f"g{i}_Whh"],
                 p[f"g{i}_bhh"])
    return _readout(symbols, x, p)


# speedup vs baseline: 2.5918x; 1.8583x over previous
import functools

import jax
import jax.numpy as jnp
from jax import lax
from jax.experimental import pallas as pl
from jax.experimental.pallas import tpu as pltpu
from jax.experimental.pallas import tpu_sc as plsc

N_NODES = 10000
N_EDGES = 2048
DIM = 128

_SC_CORES = 2
_SC_SUBCORES = 16
_NW = _SC_CORES * _SC_SUBCORES

_GB = 2 * N_EDGES
_GPW = _GB // _NW


@functools.cache
def _sc_gather_kernel():
    mesh = plsc.VectorSubcoreMesh(core_axis_name="c", subcore_axis_name="s")

    @functools.partial(
        pl.kernel,
        mesh=mesh,
        out_type=jax.ShapeDtypeStruct((_GB, DIM), jnp.float32),
        scratch_types=[
            pltpu.VMEM((_GPW,), jnp.int32),
            pltpu.VMEM((_GPW, DIM), jnp.float32),
            pltpu.SemaphoreType.DMA,
        ],
    )
    def _sc_gather(table_hbm, idx_hbm, out_hbm, idx_v, rows_v, sem):
        wid = lax.axis_index("s") * _SC_CORES + lax.axis_index("c")
        base = wid * _GPW
        pltpu.sync_copy(idx_hbm.at[pl.ds(base, _GPW)], idx_v)
        pltpu.async_copy(table_hbm.at[idx_v], rows_v, sem).wait()
        pltpu.sync_copy(rows_v, out_hbm.at[pl.ds(base, _GPW)])

    return _sc_gather


_HALF = N_NODES // _SC_CORES
_ACC_ROWS = _HALF + 8
_EPT = N_EDGES // _SC_SUBCORES
_ZPT = _ACC_ROWS // _SC_SUBCORES
_CPT = _HALF // _SC_SUBCORES


@functools.cache
def _sc_scatter_kernel():
    mesh = plsc.VectorSubcoreMesh(core_axis_name="c", subcore_axis_name="s")

    @functools.partial(
        pl.kernel,
        mesh=mesh,
        out_type=jax.ShapeDtypeStruct((N_NODES, DIM), jnp.float32),
        scratch_types=[
            pltpu.VMEM((_EPT,), jnp.int32),
            pltpu.VMEM((_EPT,), jnp.int32),
            pltpu.VMEM((_EPT, DIM), jnp.float32),
            pltpu.VMEM((_ZPT, DIM), jnp.float32),
            pltpu.VMEM_SHARED((_ACC_ROWS, DIM), jnp.float32),
        ],
    )
    def _sc_scatter(msg_hbm, tgt_hbm, out_hbm, idx_v, idx2_v, rows_v, zrows_v,
                    acc_sh):
        c = lax.axis_index("c")
        s = lax.axis_index("s")

        def _zrow(i, carry):
            for j in range(DIM // 16):
                zrows_v[i, pl.ds(j * 16, 16)] = jnp.zeros((16,), jnp.float32)
            return carry
        lax.fori_loop(0, _ZPT, _zrow, 0)
        pltpu.sync_copy(zrows_v, acc_sh.at[pl.ds(s * _ZPT, _ZPT)])

        base = s * _EPT
        pltpu.sync_copy(tgt_hbm.at[pl.ds(base, _EPT)], idx_v)
        pltpu.sync_copy(msg_hbm.at[pl.ds(base, _EPT)], rows_v)

        lo = c * _HALF
        for j in range(_EPT // 16):
            v = idx_v[pl.ds(j * 16, 16)] - lo
            inr = (v >= 0) & (v < _HALF)
            idx2_v[pl.ds(j * 16, 16)] = jnp.where(inr, v, _HALF)

        plsc.subcore_barrier()
        pltpu.sync_copy(rows_v, acc_sh.at[idx2_v], add=True)
        plsc.subcore_barrier()

        pltpu.sync_copy(acc_sh.at[pl.ds(s * _CPT, _CPT)],
                        out_hbm.at[pl.ds(lo + s * _CPT, _CPT)])

        @pl.when(s == _SC_SUBCORES - 1)
        def _():
            rem = _HALF - _SC_SUBCORES * _CPT
            pltpu.sync_copy(acc_sh.at[pl.ds(_SC_SUBCORES * _CPT, rem)],
                            out_hbm.at[pl.ds(lo + _SC_SUBCORES * _CPT, rem)])

    return _sc_scatter


def _edge_mlp_body(rows_ref, rel_ref, w1a, w1b, b1, w2, b2, out_ref):
    xs = rows_ref[:N_EDGES, :]
    xt = rows_ref[N_EDGES:, :]
    h = jnp.dot(xs, w1a[...], preferred_element_type=jnp.float32)
    h = h + jnp.dot(xt, w1b[...], preferred_element_type=jnp.float32)
    h = jnp.maximum(h + b1[...], 0.0)
    msg = jnp.dot(h, w2[...], preferred_element_type=jnp.float32) + b2[...]
    src = rel_ref[:, 0:1]
    tgt = rel_ref[:, 2:3]
    valid = ((src < N_NODES) & (tgt < N_NODES)).astype(jnp.float32)
    out_ref[...] = msg * valid


def _edge_mlp(rows, relations, W1, b1, W2, b2):
    return pl.pallas_call(
        _edge_mlp_body,
        out_shape=jax.ShapeDtypeStruct((N_EDGES, DIM), jnp.float32),
    )(rows, relations, W1[:DIM], W1[DIM:], b1.reshape(1, DIM), W2,
      b2.reshape(1, DIM))


def _gru_body(agg_ref, x_ref, wih, bih, whh, bhh, out_ref):
    gi = jnp.dot(agg_ref[...], wih[...],
                 preferred_element_type=jnp.float32) + bih[...]
    gh = jnp.dot(x_ref[...], whh[...],
                 preferred_element_type=jnp.float32) + bhh[...]
    r = jax.nn.sigmoid(gi[:, :DIM] + gh[:, :DIM])
    z = jax.nn.sigmoid(gi[:, DIM:2 * DIM] + gh[:, DIM:2 * DIM])
    n = jnp.tanh(gi[:, 2 * DIM:] + r * gh[:, 2 * DIM:])
    out_ref[...] = (1.0 - z) * n + z * x_ref[...]


def _gru(agg, x, Wih, bih, Whh, bhh):
    R = 1000
    full = lambda i: (0, 0)
    return pl.pallas_call(
        _gru_body,
        grid=(N_NODES // R,),
        in_specs=[
            pl.BlockSpec((R, DIM), lambda i: (i, 0)),
            pl.BlockSpec((R, DIM), lambda i: (i, 0)),
            pl.BlockSpec((DIM, 3 * DIM), full),
            pl.BlockSpec((1, 3 * DIM), full),
            pl.BlockSpec((DIM, 3 * DIM), full),
            pl.BlockSpec((1, 3 * DIM), full),
        ],
        out_specs=pl.BlockSpec((R, DIM), lambda i: (i, 0)),
        out_shape=jax.ShapeDtypeStruct((N_NODES, DIM), jnp.float32),
    )(agg, x, Wih, bih.reshape(1, -1), Whh, bhh.reshape(1, -1))


def _layer_norm(h, g, b):
    mu = jnp.mean(h, axis=-1, keepdims=True)
    var = jnp.mean((h - mu) ** 2, axis=-1, keepdims=True)
    return (h - mu) * jax.lax.rsqrt(var + 1e-5) * g + b


def _readout_body(sym_ref, x_ref, d1, db1, g1, c1, d2, db2, g2, c2, d3, db3,
                  out_ref):
    agg = jnp.dot(sym_ref[...], x_ref[...], preferred_element_type=jnp.float32)
    h = jnp.dot(agg, d1[...], preferred_element_type=jnp.float32) + db1[...]
    h = jnp.maximum(_layer_norm(h, g1[...], c1[...]), 0.0)
    h = jnp.dot(h, d2[...], preferred_element_type=jnp.float32) + db2[...]
    h = jnp.maximum(_layer_norm(h, g2[...], c2[...]), 0.0)
    out_ref[...] = jnp.dot(h, d3[...],
                           preferred_element_type=jnp.float32) + db3[...]


def _readout(symbols, x, p):
    B = symbols.shape[0]
    return pl.pallas_call(
        _readout_body,
        out_shape=jax.ShapeDtypeStruct((B, DIM), jnp.float32),
    )(symbols, x,
      p["D1"], p["db1"].reshape(1, -1), p["ln1_g"].reshape(1, -1),
      p["ln1_b"].reshape(1, -1),
      p["D2"], p["db2"].reshape(1, -1), p["ln2_g"].reshape(1, -1),
      p["ln2_b"].reshape(1, -1),
      p["D3"], p["db3"].reshape(1, -1))


def kernel(symbols, relations, params):
    p = params
    x = p["emb"]
    src = relations[:, 0]
    tgt = relations[:, 2]
    idx = jnp.concatenate([src, tgt], axis=0)
    for i in range(3):
        rows = _sc_gather_kernel()(x, idx)
        msg = _edge_mlp(rows, relations, p[f"g{i}_W1"], p[f"g{i}_b1"],
                        p[f"g{i}_W2"], p[f"g{i}_b2"])
        agg = _sc_scatter_kernel()(msg, tgt)
        x = _gru(agg, x, p[f"g{i}_Wih"], p[f"g{i}_bih"], p[f"g{i}_Whh"],
                 p[f"g{i}_bhh"])
    return _readout(symbols, x, p)


# speedup vs baseline: 2.7082x; 1.0449x over previous
import functools

import jax
import jax.numpy as jnp
from jax import lax
from jax.experimental import pallas as pl
from jax.experimental.pallas import tpu as pltpu
from jax.experimental.pallas import tpu_sc as plsc

N_NODES = 10000
N_EDGES = 2048
DIM = 128

_SC_CORES = 2
_SC_SUBCORES = 16
_NW = _SC_CORES * _SC_SUBCORES

_GB = 2 * N_EDGES
_GPW = _GB // _NW


@functools.cache
def _sc_gather_kernel():
    mesh = plsc.VectorSubcoreMesh(core_axis_name="c", subcore_axis_name="s")

    @functools.partial(
        pl.kernel,
        mesh=mesh,
        out_type=jax.ShapeDtypeStruct((_GB, DIM), jnp.float32),
        scratch_types=[
            pltpu.VMEM((_GPW,), jnp.int32),
            pltpu.VMEM((_GPW, DIM), jnp.float32),
            pltpu.SemaphoreType.DMA,
        ],
    )
    def _sc_gather(table_hbm, idx_hbm, out_hbm, idx_v, rows_v, sem):
        wid = lax.axis_index("s") * _SC_CORES + lax.axis_index("c")
        base = wid * _GPW
        pltpu.sync_copy(idx_hbm.at[pl.ds(base, _GPW)], idx_v)
        pltpu.async_copy(table_hbm.at[idx_v], rows_v, sem).wait()
        pltpu.sync_copy(rows_v, out_hbm.at[pl.ds(base, _GPW)])

    return _sc_gather


_HALF = N_NODES // _SC_CORES
_ACC_ROWS = _HALF + 8
_EPT = N_EDGES // _SC_SUBCORES
_ZPT = _ACC_ROWS // _SC_SUBCORES
_CPT = _HALF // _SC_SUBCORES


_ZCH = 64


@functools.cache
def _sc_scatter_kernel():
    mesh = plsc.VectorSubcoreMesh(core_axis_name="c", subcore_axis_name="s")

    @functools.partial(
        pl.kernel,
        mesh=mesh,
        out_type=jax.ShapeDtypeStruct((N_NODES, DIM), jnp.float32),
        scratch_types=[
            pltpu.VMEM((_EPT,), jnp.int32),
            pltpu.VMEM((_EPT,), jnp.int32),
            pltpu.VMEM((_EPT, DIM), jnp.float32),
            pltpu.VMEM((_ZCH, DIM), jnp.float32),
            pltpu.VMEM_SHARED((_ACC_ROWS, DIM), jnp.float32),
            pltpu.SemaphoreType.DMA,
            pltpu.SemaphoreType.DMA,
            pltpu.SemaphoreType.DMA,
        ],
    )
    def _sc_scatter(msg_hbm, tgt_hbm, out_hbm, idx_v, idx2_v, rows_v, zbuf_v,
                    acc_sh, sem_i, sem_m, sem_z):
        c = lax.axis_index("c")
        s = lax.axis_index("s")

        base = s * _EPT
        cp_i = pltpu.async_copy(tgt_hbm.at[pl.ds(base, _EPT)], idx_v, sem_i)
        cp_m = pltpu.async_copy(msg_hbm.at[pl.ds(base, _EPT)], rows_v, sem_m)

        def _zrow(i, carry):
            for j in range(DIM // 16):
                zbuf_v[i, pl.ds(j * 16, 16)] = jnp.zeros((16,), jnp.float32)
            return carry
        lax.fori_loop(0, _ZCH, _zrow, 0)
        zc = []
        for kk in range(_ZPT // _ZCH):
            zc.append(pltpu.async_copy(
                zbuf_v, acc_sh.at[pl.ds(s * _ZPT + kk * _ZCH, _ZCH)], sem_z))
        rem = _ZPT % _ZCH
        zc.append(pltpu.async_copy(
            zbuf_v.at[pl.ds(0, rem)],
            acc_sh.at[pl.ds(s * _ZPT + _ZPT - rem, rem)], sem_z))

        cp_i.wait()
        lo = c * _HALF
        for j in range(_EPT // 16):
            v = idx_v[pl.ds(j * 16, 16)] - lo
            inr = (v >= 0) & (v < _HALF)
            idx2_v[pl.ds(j * 16, 16)] = jnp.where(inr, v, _HALF)

        for z in zc:
            z.wait()
        cp_m.wait()
        plsc.subcore_barrier()
        pltpu.sync_copy(rows_v, acc_sh.at[idx2_v], add=True)
        plsc.subcore_barrier()

        pltpu.sync_copy(acc_sh.at[pl.ds(s * _CPT, _CPT)],
                        out_hbm.at[pl.ds(lo + s * _CPT, _CPT)])

        @pl.when(s == _SC_SUBCORES - 1)
        def _():
            rem = _HALF - _SC_SUBCORES * _CPT
            pltpu.sync_copy(acc_sh.at[pl.ds(_SC_SUBCORES * _CPT, rem)],
                            out_hbm.at[pl.ds(lo + _SC_SUBCORES * _CPT, rem)])

    return _sc_scatter


_EB = 512


def _edge_mlp_body(xs_ref, xt_ref, rel_ref, w1a, w1b, b1, w2, b2, out_ref):
    h = jnp.dot(xs_ref[...], w1a[...], preferred_element_type=jnp.float32)
    h = h + jnp.dot(xt_ref[...], w1b[...], preferred_element_type=jnp.float32)
    h = jnp.maximum(h + b1[...], 0.0)
    msg = jnp.dot(h, w2[...], preferred_element_type=jnp.float32) + b2[...]
    src = rel_ref[:, 0:1]
    tgt = rel_ref[:, 2:3]
    valid = ((src < N_NODES) & (tgt < N_NODES)).astype(jnp.float32)
    out_ref[...] = msg * valid


def _edge_mlp(rows, relations, W1, b1, W2, b2):
    full = lambda j: (0, 0)
    nsrc = N_EDGES // _EB
    return pl.pallas_call(
        _edge_mlp_body,
        grid=(nsrc,),
        in_specs=[
            pl.BlockSpec((_EB, DIM), lambda j: (j, 0)),
            pl.BlockSpec((_EB, DIM), lambda j: (j + nsrc, 0)),
            pl.BlockSpec((_EB, 3), lambda j: (j, 0)),
            pl.BlockSpec((DIM, DIM), full),
            pl.BlockSpec((DIM, DIM), full),
            pl.BlockSpec((1, DIM), full),
            pl.BlockSpec((DIM, DIM), full),
            pl.BlockSpec((1, DIM), full),
        ],
        out_specs=pl.BlockSpec((_EB, DIM), lambda j: (j, 0)),
        out_shape=jax.ShapeDtypeStruct((N_EDGES, DIM), jnp.float32),
    )(rows, rows, relations, W1[:DIM], W1[DIM:], b1.reshape(1, DIM), W2,
      b2.reshape(1, DIM))


def _gru_body(agg_ref, x_ref, wcat, bcat, out_ref):
    cat = jnp.concatenate([agg_ref[...], x_ref[...]], axis=1)
    g = jnp.dot(cat, wcat[...], preferred_element_type=jnp.float32) + bcat[...]
    r = jax.nn.sigmoid(g[:, :DIM])
    z = jax.nn.sigmoid(g[:, DIM:2 * DIM])
    n = jnp.tanh(g[:, 2 * DIM:3 * DIM] + r * g[:, 3 * DIM:])
    out_ref[...] = (1.0 - z) * n + z * x_ref[...]


def _gru(agg, x, Wih, bih, Whh, bhh):
    zero = jnp.zeros((DIM, DIM), jnp.float32)
    top = jnp.concatenate([Wih[:, :2 * DIM], Wih[:, 2 * DIM:], zero], axis=1)
    bot = jnp.concatenate([Whh[:, :2 * DIM], zero, Whh[:, 2 * DIM:]], axis=1)
    wcat = jnp.concatenate([top, bot], axis=0)
    bcat = jnp.concatenate(
        [bih[:2 * DIM] + bhh[:2 * DIM], bih[2 * DIM:], bhh[2 * DIM:]])
    R = 1000
    full = lambda i: (0, 0)
    return pl.pallas_call(
        _gru_body,
        grid=(N_NODES // R,),
        in_specs=[
            pl.BlockSpec((R, DIM), lambda i: (i, 0)),
            pl.BlockSpec((R, DIM), lambda i: (i, 0)),
            pl.BlockSpec((2 * DIM, 4 * DIM), full),
            pl.BlockSpec((1, 4 * DIM), full),
        ],
        out_specs=pl.BlockSpec((R, DIM), lambda i: (i, 0)),
        out_shape=jax.ShapeDtypeStruct((N_NODES, DIM), jnp.float32),
    )(agg, x, wcat, bcat.reshape(1, -1))


def _layer_norm(h, g, b):
    mu = jnp.mean(h, axis=-1, keepdims=True)
    var = jnp.mean((h - mu) ** 2, axis=-1, keepdims=True)
    return (h - mu) * jax.lax.rsqrt(var + 1e-5) * g + b


_KB = 2000


def _readout_body(symt_ref, x_ref, d1, db1, g1, c1, d2, db2, g2, c2, d3, db3,
                  out_ref, acc_ref):
    k = pl.program_id(0)

    @pl.when(k == 0)
    def _():
        acc_ref[...] = jnp.zeros_like(acc_ref)

    acc_ref[...] += lax.dot_general(
        symt_ref[...], x_ref[...], (((0,), (0,)), ((), ())),
        preferred_element_type=jnp.float32)

    @pl.when(k == pl.num_programs(0) - 1)
    def _():
        agg = acc_ref[...]
        h = jnp.dot(agg, d1[...], preferred_element_type=jnp.float32) + db1[...]
        h = jnp.maximum(_layer_norm(h, g1[...], c1[...]), 0.0)
        h = jnp.dot(h, d2[...], preferred_element_type=jnp.float32) + db2[...]
        h = jnp.maximum(_layer_norm(h, g2[...], c2[...]), 0.0)
        out_ref[...] = jnp.dot(h, d3[...],
                               preferred_element_type=jnp.float32) + db3[...]


def _readout(symbols, x, p):
    B = symbols.shape[0]
    full = lambda k: (0, 0)
    return pl.pallas_call(
        _readout_body,
        grid=(N_NODES // _KB,),
        in_specs=[
            pl.BlockSpec((_KB, B), lambda k: (k, 0)),
            pl.BlockSpec((_KB, DIM), lambda k: (k, 0)),
            pl.BlockSpec((DIM, 512), full),
            pl.BlockSpec((1, 512), full),
            pl.BlockSpec((1, 512), full),
            pl.BlockSpec((1, 512), full),
            pl.BlockSpec((512, 256), full),
            pl.BlockSpec((1, 256), full),
            pl.BlockSpec((1, 256), full),
            pl.BlockSpec((1, 256), full),
            pl.BlockSpec((256, DIM), full),
            pl.BlockSpec((1, DIM), full),
        ],
        out_specs=pl.BlockSpec((B, DIM), full),
        out_shape=jax.ShapeDtypeStruct((B, DIM), jnp.float32),
        scratch_shapes=[pltpu.VMEM((B, DIM), jnp.float32)],
    )(symbols.T, x,
      p["D1"], p["db1"].reshape(1, -1), p["ln1_g"].reshape(1, -1),
      p["ln1_b"].reshape(1, -1),
      p["D2"], p["db2"].reshape(1, -1), p["ln2_g"].reshape(1, -1),
      p["ln2_b"].reshape(1, -1),
      p["D3"], p["db3"].reshape(1, -1))


def kernel(symbols, relations, params):
    p = params
    x = p["emb"]
    src = relations[:, 0]
    tgt = relations[:, 2]
    idx = jnp.concatenate([src, tgt], axis=0)
    for i in range(3):
        rows = _sc_gather_kernel()(x, idx)
        msg = _edge_mlp(rows, relations, p[f"g{i}_W1"], p[f"g{i}_b1"],
                        p[f"g{i}_W2"], p[f"g{i}_b2"])
        agg = _sc_scatter_kernel()(msg, tgt)
        x = _gru(agg, x, p[f"g{i}_Wih"], p[f"g{i}_bih"], p[f"g{i}_Whh"],
                 p[f"g{i}_bhh"])
    return _readout(symbols, x, p)


# speedup vs baseline: 2.8061x; 1.0361x over previous
import functools

import jax
import jax.numpy as jnp
from jax import lax
from jax.experimental import pallas as pl
from jax.experimental.pallas import tpu as pltpu
from jax.experimental.pallas import tpu_sc as plsc

N_NODES = 10000
N_EDGES = 2048
DIM = 128

_SC_CORES = 2
_SC_SUBCORES = 16
_NW = _SC_CORES * _SC_SUBCORES

_GB = 2 * N_EDGES
_GPW = _GB // _NW


@functools.cache
def _sc_gather_kernel():
    mesh = plsc.VectorSubcoreMesh(core_axis_name="c", subcore_axis_name="s")

    @functools.partial(
        pl.kernel,
        mesh=mesh,
        out_type=jax.ShapeDtypeStruct((_GB, DIM), jnp.float32),
        scratch_types=[
            pltpu.VMEM((_GPW,), jnp.int32),
            pltpu.VMEM((_GPW, DIM), jnp.float32),
            pltpu.SemaphoreType.DMA,
        ],
    )
    def _sc_gather(table_hbm, idx_hbm, out_hbm, idx_v, rows_v, sem):
        wid = lax.axis_index("s") * _SC_CORES + lax.axis_index("c")
        base = wid * _GPW
        pltpu.sync_copy(idx_hbm.at[pl.ds(base, _GPW)], idx_v)
        pltpu.async_copy(table_hbm.at[idx_v], rows_v, sem).wait()
        pltpu.sync_copy(rows_v, out_hbm.at[pl.ds(base, _GPW)])

    return _sc_gather


_HALF = N_NODES // _SC_CORES
_ACC_ROWS = _HALF + 8
_EPT = N_EDGES // _SC_SUBCORES
_ZPT = _ACC_ROWS // _SC_SUBCORES
_CPT = _HALF // _SC_SUBCORES


_ZCH = 64


@functools.cache
def _sc_scatter_kernel():
    mesh = plsc.VectorSubcoreMesh(core_axis_name="c", subcore_axis_name="s")

    @functools.partial(
        pl.kernel,
        mesh=mesh,
        out_type=jax.ShapeDtypeStruct((N_NODES, DIM), jnp.float32),
        scratch_types=[
            pltpu.VMEM((_EPT,), jnp.int32),
            pltpu.VMEM((_EPT,), jnp.int32),
            pltpu.VMEM((_EPT, DIM), jnp.float32),
            pltpu.VMEM((_ZCH, DIM), jnp.float32),
            pltpu.VMEM_SHARED((_ACC_ROWS, DIM), jnp.float32),
            pltpu.SemaphoreType.DMA,
            pltpu.SemaphoreType.DMA,
            pltpu.SemaphoreType.DMA,
        ],
    )
    def _sc_scatter(msg_hbm, tgt_hbm, out_hbm, idx_v, idx2_v, rows_v, zbuf_v,
                    acc_sh, sem_i, sem_m, sem_z):
        c = lax.axis_index("c")
        s = lax.axis_index("s")

        base = s * _EPT
        cp_i = pltpu.async_copy(tgt_hbm.at[pl.ds(base, _EPT)], idx_v, sem_i)
        cp_m = pltpu.async_copy(msg_hbm.at[pl.ds(base, _EPT)], rows_v, sem_m)

        def _zrow(i, carry):
            for j in range(DIM // 16):
                zbuf_v[i, pl.ds(j * 16, 16)] = jnp.zeros((16,), jnp.float32)
            return carry
        lax.fori_loop(0, _ZCH, _zrow, 0)
        zc = []
        for kk in range(_ZPT // _ZCH):
            zc.append(pltpu.async_copy(
                zbuf_v, acc_sh.at[pl.ds(s * _ZPT + kk * _ZCH, _ZCH)], sem_z))
        rem = _ZPT % _ZCH
        zc.append(pltpu.async_copy(
            zbuf_v.at[pl.ds(0, rem)],
            acc_sh.at[pl.ds(s * _ZPT + _ZPT - rem, rem)], sem_z))

        cp_i.wait()
        lo = c * _HALF
        for j in range(_EPT // 16):
            v = idx_v[pl.ds(j * 16, 16)] - lo
            inr = (v >= 0) & (v < _HALF)
            idx2_v[pl.ds(j * 16, 16)] = jnp.where(inr, v, _HALF)

        for z in zc:
            z.wait()
        cp_m.wait()
        plsc.subcore_barrier()
        pltpu.sync_copy(rows_v, acc_sh.at[idx2_v], add=True)
        plsc.subcore_barrier()

        pltpu.sync_copy(acc_sh.at[pl.ds(s * _CPT, _CPT)],
                        out_hbm.at[pl.ds(lo + s * _CPT, _CPT)])

        @pl.when(s == _SC_SUBCORES - 1)
        def _():
            rem = _HALF - _SC_SUBCORES * _CPT
            pltpu.sync_copy(acc_sh.at[pl.ds(_SC_SUBCORES * _CPT, rem)],
                            out_hbm.at[pl.ds(lo + _SC_SUBCORES * _CPT, rem)])

    return _sc_scatter


def _edge_mlp_body(rows_ref, rel_ref, w1a, w1b, b1, w2, b2, out_ref):
    xs = rows_ref[:N_EDGES, :].astype(jnp.bfloat16)
    xt = rows_ref[N_EDGES:, :].astype(jnp.bfloat16)
    h = jnp.dot(xs, w1a[...].astype(jnp.bfloat16),
                preferred_element_type=jnp.float32)
    h = h + jnp.dot(xt, w1b[...].astype(jnp.bfloat16),
                    preferred_element_type=jnp.float32)
    h = jnp.maximum(h + b1[...], 0.0).astype(jnp.bfloat16)
    msg = jnp.dot(h, w2[...].astype(jnp.bfloat16),
                  preferred_element_type=jnp.float32) + b2[...]
    src = rel_ref[:, 0:1]
    tgt = rel_ref[:, 2:3]
    valid = ((src < N_NODES) & (tgt < N_NODES)).astype(jnp.float32)
    out_ref[...] = msg * valid


def _edge_mlp(rows, relations, W1, b1, W2, b2):
    return pl.pallas_call(
        _edge_mlp_body,
        out_shape=jax.ShapeDtypeStruct((N_EDGES, DIM), jnp.float32),
    )(rows, relations, W1[:DIM], W1[DIM:], b1.reshape(1, DIM), W2,
      b2.reshape(1, DIM))


def _gru_body(agg_ref, x_ref, wih, bih, whh, bhh, out_ref):
    gi = jnp.dot(agg_ref[...].astype(jnp.bfloat16),
                 wih[...].astype(jnp.bfloat16),
                 preferred_element_type=jnp.float32) + bih[...]
    gh = jnp.dot(x_ref[...].astype(jnp.bfloat16),
                 whh[...].astype(jnp.bfloat16),
                 preferred_element_type=jnp.float32) + bhh[...]
    r = jax.nn.sigmoid(gi[:, :DIM] + gh[:, :DIM])
    z = jax.nn.sigmoid(gi[:, DIM:2 * DIM] + gh[:, DIM:2 * DIM])
    n = jnp.tanh(gi[:, 2 * DIM:] + r * gh[:, 2 * DIM:])
    out_ref[...] = (1.0 - z) * n + z * x_ref[...]


def _gru(agg, x, Wih, bih, Whh, bhh):
    R = 1000
    full = lambda i: (0, 0)
    return pl.pallas_call(
        _gru_body,
        grid=(N_NODES // R,),
        in_specs=[
            pl.BlockSpec((R, DIM), lambda i: (i, 0)),
            pl.BlockSpec((R, DIM), lambda i: (i, 0)),
            pl.BlockSpec((DIM, 3 * DIM), full),
            pl.BlockSpec((1, 3 * DIM), full),
            pl.BlockSpec((DIM, 3 * DIM), full),
            pl.BlockSpec((1, 3 * DIM), full),
        ],
        out_specs=pl.BlockSpec((R, DIM), lambda i: (i, 0)),
        out_shape=jax.ShapeDtypeStruct((N_NODES, DIM), jnp.float32),
    )(agg, x, Wih, bih.reshape(1, -1), Whh, bhh.reshape(1, -1))


def _layer_norm(h, g, b):
    mu = jnp.mean(h, axis=-1, keepdims=True)
    var = jnp.mean((h - mu) ** 2, axis=-1, keepdims=True)
    return (h - mu) * jax.lax.rsqrt(var + 1e-5) * g + b


def _readout_body(sym_ref, x_ref, d1, db1, g1, c1, d2, db2, g2, c2, d3, db3,
                  out_ref):
    agg = jnp.dot(sym_ref[...].astype(jnp.bfloat16),
                  x_ref[...].astype(jnp.bfloat16),
                  preferred_element_type=jnp.float32)
    h = jnp.dot(agg.astype(jnp.bfloat16), d1[...].astype(jnp.bfloat16),
                preferred_element_type=jnp.float32) + db1[...]
    h = jnp.maximum(_layer_norm(h, g1[...], c1[...]), 0.0)
    h = jnp.dot(h.astype(jnp.bfloat16), d2[...].astype(jnp.bfloat16),
                preferred_element_type=jnp.float32) + db2[...]
    h = jnp.maximum(_layer_norm(h, g2[...], c2[...]), 0.0)
    out_ref[...] = jnp.dot(h.astype(jnp.bfloat16),
                           d3[...].astype(jnp.bfloat16),
                           preferred_element_type=jnp.float32) + db3[...]


def _readout(symbols, x, p):
    B = symbols.shape[0]
    return pl.pallas_call(
        _readout_body,
        out_shape=jax.ShapeDtypeStruct((B, DIM), jnp.float32),
    )(symbols, x,
      p["D1"], p["db1"].reshape(1, -1), p["ln1_g"].reshape(1, -1),
      p["ln1_b"].reshape(1, -1),
      p["D2"], p["db2"].reshape(1, -1), p["ln2_g"].reshape(1, -1),
      p["ln2_b"].reshape(1, -1),
      p["D3"], p["db3"].reshape(1, -1))


def kernel(symbols, relations, params):
    p = params
    x = p["emb"]
    src = relations[:, 0]
    tgt = relations[:, 2]
    idx = jnp.concatenate([src, tgt], axis=0)
    for i in range(3):
        rows = _sc_gather_kernel()(x, idx)
        msg = _edge_mlp(rows, relations, p[f"g{i}_W1"], p[f"g{i}_b1"],
                        p[f"g{i}_W2"], p[f"g{i}_b2"])
        agg = _sc_scatter_kernel()(msg, tgt)
        x = _gru(agg, x, p[f"g{i}_Wih"], p[f"g{i}_bih"], p[f"g{i}_Whh"],
                 p[f"g{i}_bhh"])
    return _readout(symbols, x, p)


# speedup vs baseline: 3.0180x; 1.0755x over previous
import functools

import jax
import jax.numpy as jnp
from jax import lax
from jax.experimental import pallas as pl
from jax.experimental.pallas import tpu as pltpu
from jax.experimental.pallas import tpu_sc as plsc

N_NODES = 10000
N_EDGES = 2048
DIM = 128

_SC_CORES = 2
_SC_SUBCORES = 16
_NW = _SC_CORES * _SC_SUBCORES

_GB = 2 * N_EDGES
_GPW = _GB // _NW


@functools.cache
def _sc_gather_kernel(dt):
    dt = jnp.dtype(dt)
    mesh = plsc.VectorSubcoreMesh(core_axis_name="c", subcore_axis_name="s")

    @functools.partial(
        pl.kernel,
        mesh=mesh,
        out_type=jax.ShapeDtypeStruct((_GB, DIM), dt),
        scratch_types=[
            pltpu.VMEM((_GPW,), jnp.int32),
            pltpu.VMEM((_GPW, DIM), dt),
            pltpu.SemaphoreType.DMA,
        ],
    )
    def _sc_gather(table_hbm, idx_hbm, out_hbm, idx_v, rows_v, sem):
        wid = lax.axis_index("s") * _SC_CORES + lax.axis_index("c")
        base = wid * _GPW
        pltpu.sync_copy(idx_hbm.at[pl.ds(base, _GPW)], idx_v)
        pltpu.async_copy(table_hbm.at[idx_v], rows_v, sem).wait()
        pltpu.sync_copy(rows_v, out_hbm.at[pl.ds(base, _GPW)])

    return _sc_gather


_HALF = N_NODES // _SC_CORES
_ACC_ROWS = _HALF + 8
_EPT = N_EDGES // _SC_SUBCORES
_ZPT = _ACC_ROWS // _SC_SUBCORES
_CPT = _HALF // _SC_SUBCORES


_ZCH = 64


@functools.cache
def _sc_scatter_kernel():
    mesh = plsc.VectorSubcoreMesh(core_axis_name="c", subcore_axis_name="s")

    @functools.partial(
        pl.kernel,
        mesh=mesh,
        out_type=jax.ShapeDtypeStruct((N_NODES, DIM), jnp.float32),
        scratch_types=[
            pltpu.VMEM((_EPT,), jnp.int32),
            pltpu.VMEM((_EPT,), jnp.int32),
            pltpu.VMEM((_EPT, DIM), jnp.float32),
            pltpu.VMEM((_ZCH, DIM), jnp.float32),
            pltpu.VMEM_SHARED((_ACC_ROWS, DIM), jnp.float32),
            pltpu.SemaphoreType.DMA,
            pltpu.SemaphoreType.DMA,
            pltpu.SemaphoreType.DMA,
        ],
    )
    def _sc_scatter(msg_hbm, tgt_hbm, out_hbm, idx_v, idx2_v, rows_v, zbuf_v,
                    acc_sh, sem_i, sem_m, sem_z):
        c = lax.axis_index("c")
        s = lax.axis_index("s")

        base = s * _EPT
        cp_i = pltpu.async_copy(tgt_hbm.at[pl.ds(base, _EPT)], idx_v, sem_i)
        cp_m = pltpu.async_copy(msg_hbm.at[pl.ds(base, _EPT)], rows_v, sem_m)

        def _zrow(i, carry):
            for j in range(DIM // 16):
                zbuf_v[i, pl.ds(j * 16, 16)] = jnp.zeros((16,), jnp.float32)
            return carry
        lax.fori_loop(0, _ZCH, _zrow, 0)
        zc = []
        for kk in range(_ZPT // _ZCH):
            zc.append(pltpu.async_copy(
                zbuf_v, acc_sh.at[pl.ds(s * _ZPT + kk * _ZCH, _ZCH)], sem_z))
        rem = _ZPT % _ZCH
        zc.append(pltpu.async_copy(
            zbuf_v.at[pl.ds(0, rem)],
            acc_sh.at[pl.ds(s * _ZPT + _ZPT - rem, rem)], sem_z))

        cp_i.wait()
        lo = c * _HALF
        for j in range(_EPT // 16):
            v = idx_v[pl.ds(j * 16, 16)] - lo
            inr = (v >= 0) & (v < _HALF)
            idx2_v[pl.ds(j * 16, 16)] = jnp.where(inr, v, _HALF)

        for z in zc:
            z.wait()
        cp_m.wait()
        plsc.subcore_barrier()
        pltpu.sync_copy(rows_v, acc_sh.at[idx2_v], add=True)
        plsc.subcore_barrier()

        pltpu.sync_copy(acc_sh.at[pl.ds(s * _CPT, _CPT)],
                        out_hbm.at[pl.ds(lo + s * _CPT, _CPT)])

        @pl.when(s == _SC_SUBCORES - 1)
        def _():
            rem = _HALF - _SC_SUBCORES * _CPT
            pltpu.sync_copy(acc_sh.at[pl.ds(_SC_SUBCORES * _CPT, rem)],
                            out_hbm.at[pl.ds(lo + _SC_SUBCORES * _CPT, rem)])

    return _sc_scatter


def _edge_mlp_body(rows_ref, rel_ref, w1a, w1b, b1, w2, b2, out_ref):
    xs = rows_ref[:N_EDGES, :].astype(jnp.bfloat16)
    xt = rows_ref[N_EDGES:, :].astype(jnp.bfloat16)
    h = jnp.dot(xs, w1a[...].astype(jnp.bfloat16),
                preferred_element_type=jnp.float32)
    h = h + jnp.dot(xt, w1b[...].astype(jnp.bfloat16),
                    preferred_element_type=jnp.float32)
    h = jnp.maximum(h + b1[...], 0.0).astype(jnp.bfloat16)
    msg = jnp.dot(h, w2[...].astype(jnp.bfloat16),
                  preferred_element_type=jnp.float32) + b2[...]
    src = rel_ref[:, 0:1]
    tgt = rel_ref[:, 2:3]
    valid = ((src < N_NODES) & (tgt < N_NODES)).astype(jnp.float32)
    out_ref[...] = msg * valid


def _edge_mlp(rows, relations, W1, b1, W2, b2):
    return pl.pallas_call(
        _edge_mlp_body,
        out_shape=jax.ShapeDtypeStruct((N_EDGES, DIM), jnp.float32),
    )(rows, relations, W1[:DIM], W1[DIM:], b1.reshape(1, DIM), W2,
      b2.reshape(1, DIM))


def _gru_body(agg_ref, x_ref, wih, bih, whh, bhh, out_ref):
    gi = jnp.dot(agg_ref[...].astype(jnp.bfloat16),
                 wih[...].astype(jnp.bfloat16),
                 preferred_element_type=jnp.float32) + bih[...]
    gh = jnp.dot(x_ref[...].astype(jnp.bfloat16),
                 whh[...].astype(jnp.bfloat16),
                 preferred_element_type=jnp.float32) + bhh[...]
    r = jax.nn.sigmoid(gi[:, :DIM] + gh[:, :DIM])
    z = jax.nn.sigmoid(gi[:, DIM:2 * DIM] + gh[:, DIM:2 * DIM])
    n = jnp.tanh(gi[:, 2 * DIM:] + r * gh[:, 2 * DIM:])
    x32 = x_ref[...].astype(jnp.float32)
    out_ref[...] = ((1.0 - z) * n + z * x32).astype(out_ref.dtype)


def _gru(agg, x, Wih, bih, Whh, bhh, out_dtype=jnp.float32):
    R = 2000
    full = lambda i: (0, 0)
    return pl.pallas_call(
        _gru_body,
        grid=(N_NODES // R,),
        in_specs=[
            pl.BlockSpec((R, DIM), lambda i: (i, 0)),
            pl.BlockSpec((R, DIM), lambda i: (i, 0)),
            pl.BlockSpec((DIM, 3 * DIM), full),
            pl.BlockSpec((1, 3 * DIM), full),
            pl.BlockSpec((DIM, 3 * DIM), full),
            pl.BlockSpec((1, 3 * DIM), full),
        ],
        out_specs=pl.BlockSpec((R, DIM), lambda i: (i, 0)),
        out_shape=jax.ShapeDtypeStruct((N_NODES, DIM), out_dtype),
    )(agg, x, Wih, bih.reshape(1, -1), Whh, bhh.reshape(1, -1))


def _layer_norm(h, g, b):
    mu = jnp.mean(h, axis=-1, keepdims=True)
    var = jnp.mean((h - mu) ** 2, axis=-1, keepdims=True)
    return (h - mu) * jax.lax.rsqrt(var + 1e-5) * g + b


def _readout_body(sym_ref, x_ref, d1, db1, g1, c1, d2, db2, g2, c2, d3, db3,
                  out_ref):
    agg = jnp.dot(sym_ref[...].astype(jnp.bfloat16),
                  x_ref[...].astype(jnp.bfloat16),
                  preferred_element_type=jnp.float32)
    h = jnp.dot(agg.astype(jnp.bfloat16), d1[...].astype(jnp.bfloat16),
                preferred_element_type=jnp.float32) + db1[...]
    h = jnp.maximum(_layer_norm(h, g1[...], c1[...]), 0.0)
    h = jnp.dot(h.astype(jnp.bfloat16), d2[...].astype(jnp.bfloat16),
                preferred_element_type=jnp.float32) + db2[...]
    h = jnp.maximum(_layer_norm(h, g2[...], c2[...]), 0.0)
    out_ref[...] = jnp.dot(h.astype(jnp.bfloat16),
                           d3[...].astype(jnp.bfloat16),
                           preferred_element_type=jnp.float32) + db3[...]


def _readout(symbols, x, p):
    B = symbols.shape[0]
    return pl.pallas_call(
        _readout_body,
        out_shape=jax.ShapeDtypeStruct((B, DIM), jnp.float32),
    )(symbols, x,
      p["D1"], p["db1"].reshape(1, -1), p["ln1_g"].reshape(1, -1),
      p["ln1_b"].reshape(1, -1),
      p["D2"], p["db2"].reshape(1, -1), p["ln2_g"].reshape(1, -1),
      p["ln2_b"].reshape(1, -1),
      p["D3"], p["db3"].reshape(1, -1))


def kernel(symbols, relations, params):
    p = params
    x = p["emb"]
    src = relations[:, 0]
    tgt = relations[:, 2]
    idx = jnp.concatenate([src, tgt], axis=0)
    for i in range(3):
        rows = _sc_gather_kernel(jnp.dtype(x.dtype).name)(x, idx)
        msg = _edge_mlp(rows, relations, p[f"g{i}_W1"], p[f"g{i}_b1"],
                        p[f"g{i}_W2"], p[f"g{i}_b2"])
        agg = _sc_scatter_kernel()(msg, tgt)
        odt = jnp.bfloat16 if i == 2 else jnp.float32
        x = _gru(agg, x, p[f"g{i}_Wih"], p[f"g{i}_bih"], p[f"g{i}_Whh"],
                 p[f"g{i}_bhh"], out_dtype=odt)
    return _readout(symbols, x, p)
